# baseline (speedup 1.0000x reference)
import functools

import jax
import jax.numpy as jnp
from jax import lax
from jax.experimental import pallas as pl
from jax.experimental.pallas import tpu as pltpu
from jax.experimental.pallas import tpu_sc as plsc

_B, _C, _H, _W = 32, 384, 64, 64
_D = _H * _W
_R = _B * _C
_NC, _NS = 2, 16
_NW = _NC * _NS
_RPW = _R // _NW
_CH = 8
_NCHUNK = _RPW // _CH

_mesh = plsc.VectorSubcoreMesh(core_axis_name="c", subcore_axis_name="s")


@functools.partial(
    pl.kernel,
    out_type=jax.ShapeDtypeStruct((_R, _D), jnp.float32),
    mesh=_mesh,
    scratch_types=[
        pltpu.VMEM((_NCHUNK, _CH), jnp.int32),
        pltpu.VMEM((_CH, _D), jnp.float32),
        pltpu.VMEM((_CH, _D), jnp.float32),
        pltpu.SemaphoreType.DMA,
        pltpu.SemaphoreType.DMA,
    ],
)
def _reverse_rows(x_hbm, idx_hbm, out_hbm, idx_v, buf0, buf1, sem0, sem1):
    wid = lax.axis_index("s") * _NC + lax.axis_index("c")
    base = wid * _RPW
    pltpu.sync_copy(idx_hbm.at[wid], idx_v)

    bufs = (buf0, buf1)
    sems = (sem0, sem1)
    copies = [None] * _NCHUNK
    copies[0] = pltpu.async_copy(x_hbm.at[idx_v.at[0]], bufs[0], sems[0])
    for j in range(_NCHUNK):
        copies[j].wait()
        if j + 1 < _NCHUNK:
            copies[j + 1] = pltpu.async_copy(
                x_hbm.at[idx_v.at[j + 1]], bufs[(j + 1) % 2], sems[(j + 1) % 2]
            )
        pltpu.sync_copy(bufs[j % 2], out_hbm.at[pl.ds(base + j * _CH, _CH)])


def kernel(x):
    r = jnp.arange(_R, dtype=jnp.int32)
    src = r - (r % _C) + (_C - 1) - (r % _C)
    idx = src.reshape(_NW, _NCHUNK, _CH)
    out = _reverse_rows(x.reshape(_R, _D), idx)
    return out.reshape(_B, _C, _H, _W)


# speedup vs baseline: 1.0001x; 1.0001x over previous
import functools

import jax
import jax.numpy as jnp
from jax import lax
from jax.experimental import pallas as pl
from jax.experimental.pallas import tpu as pltpu
from jax.experimental.pallas import tpu_sc as plsc

_B, _C, _H, _W = 32, 384, 64, 64
_D = _H * _W
_R = _B * _C
_NC, _NS = 2, 16
_NW = _NC * _NS
_CH = 8
_NCHUNK = _C // _CH
_NBUF = 3
_LANES = 16
_KSTEPS = _D // _LANES

_mesh = plsc.VectorSubcoreMesh(core_axis_name="c", subcore_axis_name="s")


def _reverse_chunk(buf):
    def body(k, _):
        sl = pl.ds(k * _LANES, _LANES)
        for i in range(_CH // 2):
            j = _CH - 1 - i
            va = buf[i, sl]
            vb = buf[j, sl]
            buf[i, sl] = vb
            buf[j, sl] = va
        return 0

    lax.fori_loop(0, _KSTEPS, body, 0)


@functools.partial(
    pl.kernel,
    out_type=jax.ShapeDtypeStruct((_R, _D), jnp.float32),
    mesh=_mesh,
    scratch_types=[
        pltpu.VMEM((_CH, _D), jnp.float32),
        pltpu.VMEM((_CH, _D), jnp.float32),
        pltpu.VMEM((_CH, _D), jnp.float32),
        pltpu.SemaphoreType.DMA,
        pltpu.SemaphoreType.DMA,
        pltpu.SemaphoreType.DMA,
        pltpu.SemaphoreType.DMA,
        pltpu.SemaphoreType.DMA,
        pltpu.SemaphoreType.DMA,
    ],
)
def _reverse_rows(x_hbm, out_hbm, buf0, buf1, buf2,
                  rs0, rs1, rs2, ws0, ws1, ws2):
    wid = lax.axis_index("s") * _NC + lax.axis_index("c")
    base = wid * _C
    bufs = (buf0, buf1, buf2)
    rsems = (rs0, rs1, rs2)
    wsems = (ws0, ws1, ws2)

    def src_slice(j):
        return pl.ds(base + _C - (j + 1) * _CH, _CH)

    def dst_slice(j):
        return pl.ds(base + j * _CH, _CH)

    reads = [None] * _NCHUNK
    writes = [None] * _NCHUNK
    for j in range(min(2, _NCHUNK)):
        reads[j] = pltpu.async_copy(x_hbm.at[src_slice(j)], bufs[j % _NBUF],
                                    rsems[j % _NBUF])
    for j in range(_NCHUNK):
        b = j % _NBUF
        reads[j].wait()
        _reverse_chunk(bufs[b])
        writes[j] = pltpu.async_copy(bufs[b], out_hbm.at[dst_slice(j)],
                                     wsems[b])
        nxt = j + 2
        if nxt < _NCHUNK:
            nb = nxt % _NBUF
            if nxt >= _NBUF:
                writes[nxt - _NBUF].wait()
            reads[nxt] = pltpu.async_copy(x_hbm.at[src_slice(nxt)], bufs[nb],
                                          rsems[nb])
    for j in range(_NCHUNK - _NBUF, _NCHUNK):
        if j >= 0:
            writes[j].wait()


def kernel(x):
    out = _reverse_rows(x.reshape(_R, _D))
    return out.reshape(_B, _C, _H, _W)


# speedup vs baseline: 1.7340x; 1.7340x over previous
import functools

import jax
import jax.numpy as jnp
from jax import lax
from jax.experimental import pallas as pl
from jax.experimental.pallas import tpu as pltpu
from jax.experimental.pallas import tpu_sc as plsc

_B, _C, _H, _W = 32, 384, 64, 64
_R = _B * _C
_NC, _NS = 2, 16
_NW = _NC * _NS
_CH = 4
_NCHUNK = _C // _CH
_NBUF = 3
_LANES = 16
_KSTEPS = _H * (_W // _LANES)

_mesh = plsc.VectorSubcoreMesh(core_axis_name="c", subcore_axis_name="s")


def _reverse_chunk(buf):
    def body(k, _):
        r = k // (_W // _LANES)
        c0 = (k % (_W // _LANES)) * _LANES
        sl = pl.ds(c0, _LANES)
        for i in range(_CH // 2):
            j = _CH - 1 - i
            va = buf[i, r, sl]
            vb = buf[j, r, sl]
            buf[i, r, sl] = vb
            buf[j, r, sl] = va
        return 0

    lax.fori_loop(0, _KSTEPS, body, 0)


@functools.partial(
    pl.kernel,
    out_type=jax.ShapeDtypeStruct((_R, _H, _W), jnp.float32),
    mesh=_mesh,
    scratch_types=[
        pltpu.VMEM((_CH, _H, _W), jnp.float32),
        pltpu.VMEM((_CH, _H, _W), jnp.float32),
        pltpu.VMEM((_CH, _H, _W), jnp.float32),
        pltpu.SemaphoreType.DMA,
        pltpu.SemaphoreType.DMA,
        pltpu.SemaphoreType.DMA,
        pltpu.SemaphoreType.DMA,
        pltpu.SemaphoreType.DMA,
        pltpu.SemaphoreType.DMA,
    ],
)
def _reverse_rows(x_hbm, out_hbm, buf0, buf1, buf2,
                  rs0, rs1, rs2, ws0, ws1, ws2):
    wid = lax.axis_index("s") * _NC + lax.axis_index("c")
    base = wid * _C
    bufs = (buf0, buf1, buf2)
    rsems = (rs0, rs1, rs2)
    wsems = (ws0, ws1, ws2)

    def src_slice(j):
        return pl.ds(base + _C - (j + 1) * _CH, _CH)

    def dst_slice(j):
        return pl.ds(base + j * _CH, _CH)

    reads = [None] * _NCHUNK
    writes = [None] * _NCHUNK
    for j in range(min(2, _NCHUNK)):
        reads[j] = pltpu.async_copy(x_hbm.at[src_slice(j)], bufs[j % _NBUF],
                                    rsems[j % _NBUF])
    for j in range(_NCHUNK):
        b = j % _NBUF
        reads[j].wait()
        _reverse_chunk(bufs[b])
        writes[j] = pltpu.async_copy(bufs[b], out_hbm.at[dst_slice(j)],
                                     wsems[b])
        nxt = j + 2
        if nxt < _NCHUNK:
            nb = nxt % _NBUF
            if nxt >= _NBUF:
                writes[nxt - _NBUF].wait()
            reads[nxt] = pltpu.async_copy(x_hbm.at[src_slice(nxt)], bufs[nb],
                                          rsems[nb])
    for j in range(_NCHUNK - _NBUF, _NCHUNK):
        if j >= 0:
            writes[j].wait()


def kernel(x):
    out = _reverse_rows(x.reshape(_R, _H, _W))
    return out.reshape(_B, _C, _H, _W)


# speedup vs baseline: 6.7037x; 3.8659x over previous
import functools

import jax
import jax.numpy as jnp
from jax import lax
from jax.experimental import pallas as pl
from jax.experimental.pallas import tpu as pltpu
from jax.experimental.pallas import tpu_sc as plsc

_B, _C, _H, _W = 32, 384, 64, 64
_R = _B * _H * _W
_NC, _NS = 2, 16
_NW = _NC * _NS
_RPW = _R // _NW
_RCH = 64
_NCHUNK = _RPW // _RCH
_NBUF = 3
_LANES = 16
_NK = _C // _LANES

_mesh = plsc.VectorSubcoreMesh(core_axis_name="c", subcore_axis_name="s")


def _reverse_lanes(buf):
    def body(r, _):
        for k in range(_NK // 2):
            k2 = _NK - 1 - k
            sa = pl.ds(k * _LANES, _LANES)
            sb = pl.ds(k2 * _LANES, _LANES)
            va = buf[r, sa]
            vb = buf[r, sb]
            buf[r, sa] = lax.rev(vb, (0,))
            buf[r, sb] = lax.rev(va, (0,))
        return 0

    lax.fori_loop(0, _RCH, body, 0)


@functools.partial(
    pl.kernel,
    out_type=jax.ShapeDtypeStruct((_R, _C), jnp.float32),
    mesh=_mesh,
    scratch_types=[
        pltpu.VMEM((_RCH, _C), jnp.float32),
        pltpu.VMEM((_RCH, _C), jnp.float32),
        pltpu.VMEM((_RCH, _C), jnp.float32),
        pltpu.SemaphoreType.DMA,
        pltpu.SemaphoreType.DMA,
        pltpu.SemaphoreType.DMA,
        pltpu.SemaphoreType.DMA,
        pltpu.SemaphoreType.DMA,
        pltpu.SemaphoreType.DMA,
    ],
)
def _reverse_rows(x_hbm, out_hbm, buf0, buf1, buf2,
                  rs0, rs1, rs2, ws0, ws1, ws2):
    wid = lax.axis_index("s") * _NC + lax.axis_index("c")
    base = wid * _RPW
    bufs = (buf0, buf1, buf2)
    rsems = (rs0, rs1, rs2)
    wsems = (ws0, ws1, ws2)

    def rows(j):
        return pl.ds(base + j * _RCH, _RCH)

    reads = [None] * _NCHUNK
    writes = [None] * _NCHUNK
    for j in range(min(2, _NCHUNK)):
        reads[j] = pltpu.async_copy(x_hbm.at[rows(j)], bufs[j % _NBUF],
                                    rsems[j % _NBUF])
    for j in range(_NCHUNK):
        b = j % _NBUF
        reads[j].wait()
        _reverse_lanes(bufs[b])
        writes[j] = pltpu.async_copy(bufs[b], out_hbm.at[rows(j)],
                                     wsems[b])
        nxt = j + 2
        if nxt < _NCHUNK:
            nb = nxt % _NBUF
            if nxt >= _NBUF:
                writes[nxt - _NBUF].wait()
            reads[nxt] = pltpu.async_copy(x_hbm.at[rows(nxt)], bufs[nb],
                                          rsems[nb])
    for j in range(_NCHUNK - _NBUF, _NCHUNK):
        if j >= 0:
            writes[j].wait()


def kernel(x):
    xt = jnp.transpose(x, (0, 2, 3, 1)).reshape(_R, _C)
    out = _reverse_rows(xt)
    return jnp.transpose(out.reshape(_B, _H, _W, _C), (0, 3, 1, 2))
